# baseline (speedup 1.0000x reference)
import functools

import jax
import jax.numpy as jnp
from jax import lax
from jax.experimental import pallas as pl
from jax.experimental.pallas import tpu as pltpu
from jax.experimental.pallas import tpu_sc as plsc

NC = 2
NS = 16
NW = NC * NS
SUB = 128
CH = 2 * SUB
F32 = jnp.float32


def _ceil_to(v, m):
    return -(-v // m) * m


def _mesh():
    return plsc.VectorSubcoreMesh(core_axis_name="c", subcore_axis_name="s")


SEG_G = 20


def _segsum(table, src4, dst4, n_nodes_pad):
    n_groups = src4.shape[1]
    rows_per_tile = n_nodes_pad // NS

    def body(table_h, src_h, dst_h, out_h, sidx, didx, rows, accum, semg, sems):
        cid = lax.axis_index("c")
        sid = lax.axis_index("s")
        wid = sid * NC + cid

        def zrow(i, carry):
            for j in range(8):
                rows[0, i, pl.ds(16 * j, 16)] = jnp.zeros((16,), F32)
            return carry

        lax.fori_loop(0, SUB, zrow, 0)
        base = sid * rows_per_tile
        off = 0
        while off < rows_per_tile:
            n = min(SUB, rows_per_tile - off)
            pltpu.sync_copy(rows.at[0, pl.ds(0, n)], accum.at[pl.ds(base + off, n)])
            off += n
        plsc.subcore_barrier()

        def group(g, carry):
            pltpu.sync_copy(src_h.at[wid, g], sidx)
            pltpu.sync_copy(dst_h.at[wid, g], didx)
            gd = [None] * SEG_G
            sd = [None] * SEG_G
            for b in range(SEG_G):
                if b >= 2:
                    sd[b - 2].wait()
                gd[b] = pltpu.async_copy(table_h.at[sidx.at[b]],
                                         rows.at[b % 2], semg)
                if b >= 1:
                    gd[b - 1].wait()
                    sd[b - 1] = pltpu.async_copy(rows.at[(b - 1) % 2],
                                                 accum.at[didx.at[b - 1]],
                                                 sems, add=True)
            gd[SEG_G - 1].wait()
            sd[SEG_G - 1] = pltpu.async_copy(rows.at[(SEG_G - 1) % 2],
                                             accum.at[didx.at[SEG_G - 1]],
                                             sems, add=True)
            sd[SEG_G - 2].wait()
            sd[SEG_G - 1].wait()
            return carry

        lax.fori_loop(0, n_groups, group, 0)
        plsc.subcore_barrier()
        pltpu.sync_copy(accum.at[pl.ds(base, rows_per_tile)],
                        out_h.at[cid, pl.ds(base, rows_per_tile)])

    return pl.kernel(
        body,
        out_type=jax.ShapeDtypeStruct((NC, n_nodes_pad, 128), F32),
        mesh=_mesh(),
        scratch_types=[
            pltpu.VMEM((SEG_G, SUB), jnp.int32),
            pltpu.VMEM((SEG_G, SUB), jnp.int32),
            pltpu.VMEM((2, SUB, 128), F32),
            pltpu.VMEM_SHARED((n_nodes_pad, 128), F32),
            pltpu.SemaphoreType.DMA,
            pltpu.SemaphoreType.DMA,
        ],
    )(table, src4, dst4)


def _pool(table, idx4, n_rows_pad, n_perms, k_slots):
    n_chunks = n_rows_pad // CH
    per_worker = -(-n_chunks // NW)
    P, K = n_perms, k_slots

    def body(table_h, idx_h, out_h, ibuf, zbuf, sem):
        cid = lax.axis_index("c")
        sid = lax.axis_index("s")
        wid = sid * NC + cid

        def chunk(t, carry):
            ci = t * NW + wid

            @pl.when(ci < n_chunks)
            def _():
                descs = []
                for p in range(P):
                    for k in range(K):
                        descs.append(pltpu.async_copy(idx_h.at[p, k, ci],
                                                      ibuf.at[p, k], sem))
                for d in descs:
                    d.wait()
                for k in range(K):
                    descs = []
                    for p in range(P):
                        for j in range(2):
                            descs.append(pltpu.async_copy(
                                table_h.at[ibuf.at[p, k, j]],
                                zbuf.at[p, pl.ds(SUB * j, SUB)],
                                sem, add=(k > 0)))
                    for d in descs:
                        d.wait()
                descs = []
                for p in range(P):
                    descs.append(pltpu.async_copy(
                        zbuf.at[p], out_h.at[p, pl.ds(ci * CH, CH)], sem))
                for d in descs:
                    d.wait()

            return carry

        lax.fori_loop(0, per_worker, chunk, 0)

    return pl.kernel(
        body,
        out_type=jax.ShapeDtypeStruct((P, n_rows_pad, 128), F32),
        mesh=_mesh(),
        scratch_types=[
            pltpu.VMEM((P, K, 2, SUB), jnp.int32),
            pltpu.VMEM((P, CH, 128), F32),
            pltpu.SemaphoreType.DMA,
        ],
    )(table, idx4)


def _tc_layer(xin, aggs, wself, wneigh, bias):
    n = xin.shape[0]
    blk = 1000

    def kfn(x_ref, a_ref, ws_ref, wn_ref, b_ref, o_ref):
        agg = a_ref[0] + a_ref[1]
        acc = jnp.dot(x_ref[...], ws_ref[...], preferred_element_type=F32)
        acc = acc + jnp.dot(agg, wn_ref[...], preferred_element_type=F32)
        o_ref[...] = jnp.maximum(acc + b_ref[...], 0.0)

    return pl.pallas_call(
        kfn,
        grid=(n // blk,),
        in_specs=[
            pl.BlockSpec((blk, 128), lambda i: (i, 0)),
            pl.BlockSpec((NC, blk, 128), lambda i: (0, i, 0)),
            pl.BlockSpec((128, 128), lambda i: (0, 0)),
            pl.BlockSpec((128, 128), lambda i: (0, 0)),
            pl.BlockSpec((1, 128), lambda i: (0, 0)),
        ],
        out_specs=pl.BlockSpec((blk, 128), lambda i: (i, 0)),
        out_shape=jax.ShapeDtypeStruct((n, 128), F32),
    )(xin, aggs, wself, wneigh, bias.reshape(1, 128))


def _tc_heads(h, aggs, ws2, wn2, b2, wa1, ba1, wa2, ba2, wb1, wg1, wp1, wi1):
    n = h.shape[0]
    blk = 1000

    def kfn(h_ref, a_ref, ws_ref, wn_ref, b_ref, wa1_ref, ba1_ref, wa2_ref,
            ba2_ref, wb1_ref, wg1_ref, wp1_ref, wi1_ref,
            at_ref, tb_ref, tg_ref, tp_ref, ti_ref):
        agg = a_ref[0] + a_ref[1]
        nr = jnp.dot(h_ref[...], ws_ref[...], preferred_element_type=F32)
        nr = nr + jnp.dot(agg, wn_ref[...], preferred_element_type=F32)
        nr = jnp.maximum(nr + b_ref[...], 0.0)
        t = jnp.maximum(jnp.dot(nr, wa1_ref[...], preferred_element_type=F32)
                        + ba1_ref[...], 0.0)
        at_ref[...] = jnp.dot(t, wa2_ref[...], preferred_element_type=F32) + ba2_ref[...]
        for j in range(2):
            tb_ref[j] = jnp.dot(nr, wb1_ref[pl.ds(128 * j, 128)],
                                preferred_element_type=F32)
        for j in range(3):
            tg_ref[j] = jnp.dot(nr, wg1_ref[pl.ds(128 * j, 128)],
                                preferred_element_type=F32)
        for j in range(4):
            tp_ref[j] = jnp.dot(nr, wp1_ref[pl.ds(128 * j, 128)],
                                preferred_element_type=F32)
        for j in range(4):
            ti_ref[j] = jnp.dot(nr, wi1_ref[pl.ds(128 * j, 128)],
                                preferred_element_type=F32)

    full = lambda shape: pl.BlockSpec(shape, lambda i: tuple(0 for _ in shape))
    return pl.pallas_call(
        kfn,
        grid=(n // blk,),
        in_specs=[
            pl.BlockSpec((blk, 128), lambda i: (i, 0)),
            pl.BlockSpec((NC, blk, 128), lambda i: (0, i, 0)),
            full((128, 128)), full((128, 128)), full((1, 128)),
            full((128, 128)), full((1, 128)), full((128, 2)), full((1, 2)),
            full((256, 128)), full((384, 128)), full((512, 128)), full((512, 128)),
        ],
        out_specs=[
            pl.BlockSpec((blk, 2), lambda i: (i, 0)),
            pl.BlockSpec((2, blk, 128), lambda i: (0, i, 0)),
            pl.BlockSpec((3, blk, 128), lambda i: (0, i, 0)),
            pl.BlockSpec((4, blk, 128), lambda i: (0, i, 0)),
            pl.BlockSpec((4, blk, 128), lambda i: (0, i, 0)),
        ],
        out_shape=[
            jax.ShapeDtypeStruct((n, 2), F32),
            jax.ShapeDtypeStruct((2, n, 128), F32),
            jax.ShapeDtypeStruct((3, n, 128), F32),
            jax.ShapeDtypeStruct((4, n, 128), F32),
            jax.ShapeDtypeStruct((4, n, 128), F32),
        ],
    )(h, aggs, ws2, wn2, b2.reshape(1, 128), wa1, ba1.reshape(1, 128), wa2,
      ba2.reshape(1, 2), wb1, wg1, wp1, wi1)


def _tc_readout(z, w2, b1v, b2v, n_rows):
    P, rp, _ = z.shape
    out_d = w2.shape[1]
    blk = 512

    def kfn(z_ref, w2_ref, b1_ref, b2_ref, o_ref):
        acc = None
        for p in range(P):
            y = jnp.maximum(z_ref[p] + b1_ref[...], 0.0)
            yy = jnp.dot(y, w2_ref[...], preferred_element_type=F32)
            acc = yy if acc is None else acc + yy
        o_ref[...] = acc + float(P) * b2_ref[...]

    out = pl.pallas_call(
        kfn,
        grid=(rp // blk,),
        in_specs=[
            pl.BlockSpec((P, blk, 128), lambda i: (0, i, 0)),
            pl.BlockSpec((128, out_d), lambda i: (0, 0)),
            pl.BlockSpec((1, 128), lambda i: (0, 0)),
            pl.BlockSpec((1, out_d), lambda i: (0, 0)),
        ],
        out_specs=pl.BlockSpec((blk, out_d), lambda i: (i, 0)),
        out_shape=jax.ShapeDtypeStruct((rp, out_d), F32),
    )(z, w2, b1v.reshape(1, 128), b2v.reshape(1, out_d))
    return out[:n_rows]


def _readout(table, inter, perms, n_nodes, w2, b1v, b2v):
    r, _ = inter.shape
    P = len(perms)
    K = len(perms[0])
    rp = _ceil_to(r, 512)
    cols = []
    for perm in perms:
        cols.append(jnp.stack([inter[:, perm[j]] + j * n_nodes for j in range(K)]))
    idx = jnp.stack(cols)
    idx = jnp.pad(idx, ((0, 0), (0, 0), (0, rp - r)))
    idx4 = idx.reshape(P, K, rp // CH, 2, SUB)
    z = _pool(table, idx4, rp, P, K)
    return _tc_readout(z, w2, b1v, b2v, r)


def kernel(x, edge_index, bonds, angles, propers, impropers, Ws1, Wn1, b1,
           Ws2, Wn2, b2, Wa1, ba1, Wa2, ba2, Wb1, bb1, Wb2, bb2, Wg1, bg1,
           Wg2, bg2, Wp1, bp1, Wp2, bp2, Wi1, bi1, Wi2, bi2):
    n = x.shape[0]
    e = edge_index.shape[1]
    n_pad = _ceil_to(n + 1, NS * 8)
    e_pad = _ceil_to(e, SEG_G * SUB * NW)

    src = jnp.pad(edge_index[0], (0, e_pad - e))
    dst = jnp.pad(edge_index[1], (0, e_pad - e), constant_values=n)
    n_groups = e_pad // (SEG_G * SUB * NW)
    src4 = src.reshape(NW, n_groups, SEG_G, SUB)
    dst4 = dst.reshape(NW, n_groups, SEG_G, SUB)

    agg1 = _segsum(x, src4, dst4, n_pad)[:, :n]
    h = _tc_layer(x, agg1, Ws1, Wn1, b1)
    agg2 = _segsum(h, src4, dst4, n_pad)[:, :n]
    atoms, tb, tg, tp, ti = _tc_heads(h, agg2, Ws2, Wn2, b2, Wa1, ba1, Wa2,
                                      ba2, Wb1, Wg1, Wp1, Wi1)

    bonds_out = _readout(tb.reshape(2 * n, 128), bonds, [(0, 1), (1, 0)],
                         n, Wb2, bb1, bb2)
    angles_out = _readout(tg.reshape(3 * n, 128), angles,
                          [(0, 1, 2), (2, 1, 0)], n, Wg2, bg1, bg2)
    propers_out = _readout(tp.reshape(4 * n, 128), propers,
                           [(0, 1, 2, 3), (3, 2, 1, 0)], n, Wp2, bp1, bp2)
    imp_perms = [(0, 1, 2, 3), (2, 1, 3, 0), (3, 1, 0, 2)]
    impropers_out = _readout(ti.reshape(4 * n, 128), impropers, imp_perms,
                             n, Wi2, bi1, bi2)
    return (atoms, bonds_out, angles_out, propers_out, impropers_out)


# speedup vs baseline: 1.0349x; 1.0349x over previous
import functools

import jax
import jax.numpy as jnp
from jax import lax
from jax.experimental import pallas as pl
from jax.experimental.pallas import tpu as pltpu
from jax.experimental.pallas import tpu_sc as plsc

NC = 2
NS = 16
NW = NC * NS
SUB = 128
CH = 2 * SUB
F32 = jnp.float32


def _ceil_to(v, m):
    return -(-v // m) * m


def _mesh():
    return plsc.VectorSubcoreMesh(core_axis_name="c", subcore_axis_name="s")


SEG_G = 20


def _segsum(table, src4, dst4, n_nodes_pad):
    n_groups = src4.shape[0]
    rows_per_tile = n_nodes_pad // NS

    def body(table_h, src_h, dst_h, out_h, sidx, didx, rows, accum, semg, sems):
        cid = lax.axis_index("c")
        sid = lax.axis_index("s")
        wid = sid * NC + cid

        def zrow(i, carry):
            for j in range(8):
                rows[0, i, pl.ds(16 * j, 16)] = jnp.zeros((16,), F32)
            return carry

        lax.fori_loop(0, SUB, zrow, 0)
        base = sid * rows_per_tile
        off = 0
        while off < rows_per_tile:
            n = min(SUB, rows_per_tile - off)
            pltpu.sync_copy(rows.at[0, pl.ds(0, n)], accum.at[pl.ds(base + off, n)])
            off += n
        plsc.subcore_barrier()

        def group(g, carry):
            pltpu.sync_copy(src_h.at[g, wid], sidx)
            pltpu.sync_copy(dst_h.at[g, wid], didx)
            gd = [None] * SEG_G
            sd = [None] * SEG_G
            for b in range(SEG_G):
                if b >= 2:
                    sd[b - 2].wait()
                gd[b] = pltpu.async_copy(table_h.at[sidx.at[b]],
                                         rows.at[b % 2], semg)
                if b >= 1:
                    gd[b - 1].wait()
                    sd[b - 1] = pltpu.async_copy(rows.at[(b - 1) % 2],
                                                 accum.at[didx.at[b - 1]],
                                                 sems, add=True)
            gd[SEG_G - 1].wait()
            sd[SEG_G - 1] = pltpu.async_copy(rows.at[(SEG_G - 1) % 2],
                                             accum.at[didx.at[SEG_G - 1]],
                                             sems, add=True)
            sd[SEG_G - 2].wait()
            sd[SEG_G - 1].wait()
            return carry

        lax.fori_loop(0, n_groups, group, 0)
        plsc.subcore_barrier()
        pltpu.sync_copy(accum.at[pl.ds(base, rows_per_tile)],
                        out_h.at[cid, pl.ds(base, rows_per_tile)])

    return pl.kernel(
        body,
        out_type=jax.ShapeDtypeStruct((NC, n_nodes_pad, 128), F32),
        mesh=_mesh(),
        scratch_types=[
            pltpu.VMEM((SEG_G, SUB), jnp.int32),
            pltpu.VMEM((SEG_G, SUB), jnp.int32),
            pltpu.VMEM((2, SUB, 128), F32),
            pltpu.VMEM_SHARED((n_nodes_pad, 128), F32),
            pltpu.SemaphoreType.DMA,
            pltpu.SemaphoreType.DMA,
        ],
    )(table, src4, dst4)


def _pool(table, idx4, n_rows_pad, n_perms, k_slots):
    n_chunks = n_rows_pad // CH
    per_worker = -(-n_chunks // NW)
    P, K = n_perms, k_slots

    def body(table_h, idx_h, out_h, ibuf, zbuf, sem):
        cid = lax.axis_index("c")
        sid = lax.axis_index("s")
        wid = sid * NC + cid

        def chunk(t, carry):
            ci = t * NW + wid

            @pl.when(ci < n_chunks)
            def _():
                descs = []
                for p in range(P):
                    for k in range(K):
                        descs.append(pltpu.async_copy(idx_h.at[p, k, ci],
                                                      ibuf.at[p, k], sem))
                for d in descs:
                    d.wait()
                for k in range(K):
                    descs = []
                    for p in range(P):
                        for j in range(2):
                            descs.append(pltpu.async_copy(
                                table_h.at[ibuf.at[p, k, j]],
                                zbuf.at[p, pl.ds(SUB * j, SUB)],
                                sem, add=(k > 0)))
                    for d in descs:
                        d.wait()
                descs = []
                for p in range(P):
                    descs.append(pltpu.async_copy(
                        zbuf.at[p], out_h.at[p, pl.ds(ci * CH, CH)], sem))
                for d in descs:
                    d.wait()

            return carry

        lax.fori_loop(0, per_worker, chunk, 0)

    return pl.kernel(
        body,
        out_type=jax.ShapeDtypeStruct((P, n_rows_pad, 128), F32),
        mesh=_mesh(),
        scratch_types=[
            pltpu.VMEM((P, K, 2, SUB), jnp.int32),
            pltpu.VMEM((P, CH, 128), F32),
            pltpu.SemaphoreType.DMA,
        ],
    )(table, idx4)


def _tc_layer(xin, aggs, wself, wneigh, bias):
    n = xin.shape[0]
    blk = 1000

    def kfn(x_ref, a_ref, ws_ref, wn_ref, b_ref, o_ref):
        agg = a_ref[0] + a_ref[1]
        acc = jnp.dot(x_ref[...], ws_ref[...], preferred_element_type=F32)
        acc = acc + jnp.dot(agg, wn_ref[...], preferred_element_type=F32)
        o_ref[...] = jnp.maximum(acc + b_ref[...], 0.0)

    return pl.pallas_call(
        kfn,
        grid=(n // blk,),
        in_specs=[
            pl.BlockSpec((blk, 128), lambda i: (i, 0)),
            pl.BlockSpec((NC, blk, 128), lambda i: (0, i, 0)),
            pl.BlockSpec((128, 128), lambda i: (0, 0)),
            pl.BlockSpec((128, 128), lambda i: (0, 0)),
            pl.BlockSpec((1, 128), lambda i: (0, 0)),
        ],
        out_specs=pl.BlockSpec((blk, 128), lambda i: (i, 0)),
        out_shape=jax.ShapeDtypeStruct((n, 128), F32),
    )(xin, aggs, wself, wneigh, bias.reshape(1, 128))


def _tc_heads(h, aggs, ws2, wn2, b2, wa1, ba1, wa2, ba2, wb1, wg1, wp1, wi1):
    n = h.shape[0]
    blk = 1000

    def kfn(h_ref, a_ref, ws_ref, wn_ref, b_ref, wa1_ref, ba1_ref, wa2_ref,
            ba2_ref, wb1_ref, wg1_ref, wp1_ref, wi1_ref,
            at_ref, tb_ref, tg_ref, tp_ref, ti_ref):
        agg = a_ref[0] + a_ref[1]
        nr = jnp.dot(h_ref[...], ws_ref[...], preferred_element_type=F32)
        nr = nr + jnp.dot(agg, wn_ref[...], preferred_element_type=F32)
        nr = jnp.maximum(nr + b_ref[...], 0.0)
        t = jnp.maximum(jnp.dot(nr, wa1_ref[...], preferred_element_type=F32)
                        + ba1_ref[...], 0.0)
        at_ref[...] = jnp.dot(t, wa2_ref[...], preferred_element_type=F32) + ba2_ref[...]
        for j in range(2):
            tb_ref[j] = jnp.dot(nr, wb1_ref[pl.ds(128 * j, 128)],
                                preferred_element_type=F32)
        for j in range(3):
            tg_ref[j] = jnp.dot(nr, wg1_ref[pl.ds(128 * j, 128)],
                                preferred_element_type=F32)
        for j in range(4):
            tp_ref[j] = jnp.dot(nr, wp1_ref[pl.ds(128 * j, 128)],
                                preferred_element_type=F32)
        for j in range(4):
            ti_ref[j] = jnp.dot(nr, wi1_ref[pl.ds(128 * j, 128)],
                                preferred_element_type=F32)

    full = lambda shape: pl.BlockSpec(shape, lambda i: tuple(0 for _ in shape))
    return pl.pallas_call(
        kfn,
        grid=(n // blk,),
        in_specs=[
            pl.BlockSpec((blk, 128), lambda i: (i, 0)),
            pl.BlockSpec((NC, blk, 128), lambda i: (0, i, 0)),
            full((128, 128)), full((128, 128)), full((1, 128)),
            full((128, 128)), full((1, 128)), full((128, 2)), full((1, 2)),
            full((256, 128)), full((384, 128)), full((512, 128)), full((512, 128)),
        ],
        out_specs=[
            pl.BlockSpec((blk, 2), lambda i: (i, 0)),
            pl.BlockSpec((2, blk, 128), lambda i: (0, i, 0)),
            pl.BlockSpec((3, blk, 128), lambda i: (0, i, 0)),
            pl.BlockSpec((4, blk, 128), lambda i: (0, i, 0)),
            pl.BlockSpec((4, blk, 128), lambda i: (0, i, 0)),
        ],
        out_shape=[
            jax.ShapeDtypeStruct((n, 2), F32),
            jax.ShapeDtypeStruct((2, n, 128), F32),
            jax.ShapeDtypeStruct((3, n, 128), F32),
            jax.ShapeDtypeStruct((4, n, 128), F32),
            jax.ShapeDtypeStruct((4, n, 128), F32),
        ],
    )(h, aggs, ws2, wn2, b2.reshape(1, 128), wa1, ba1.reshape(1, 128), wa2,
      ba2.reshape(1, 2), wb1, wg1, wp1, wi1)


def _tc_readout(z, w2, b1v, b2v, n_rows):
    P, rp, _ = z.shape
    out_d = w2.shape[1]
    blk = 512

    def kfn(z_ref, w2_ref, b1_ref, b2_ref, o_ref):
        acc = None
        for p in range(P):
            y = jnp.maximum(z_ref[p] + b1_ref[...], 0.0)
            yy = jnp.dot(y, w2_ref[...], preferred_element_type=F32)
            acc = yy if acc is None else acc + yy
        o_ref[...] = acc + float(P) * b2_ref[...]

    out = pl.pallas_call(
        kfn,
        grid=(rp // blk,),
        in_specs=[
            pl.BlockSpec((P, blk, 128), lambda i: (0, i, 0)),
            pl.BlockSpec((128, out_d), lambda i: (0, 0)),
            pl.BlockSpec((1, 128), lambda i: (0, 0)),
            pl.BlockSpec((1, out_d), lambda i: (0, 0)),
        ],
        out_specs=pl.BlockSpec((blk, out_d), lambda i: (i, 0)),
        out_shape=jax.ShapeDtypeStruct((rp, out_d), F32),
    )(z, w2, b1v.reshape(1, 128), b2v.reshape(1, out_d))
    return out[:n_rows]


def _readout(table, inter, perms, n_nodes, w2, b1v, b2v):
    r, _ = inter.shape
    P = len(perms)
    K = len(perms[0])
    rp = _ceil_to(r, 512)
    cols = []
    for perm in perms:
        cols.append(jnp.stack([inter[:, perm[j]] + j * n_nodes for j in range(K)]))
    idx = jnp.stack(cols)
    idx = jnp.pad(idx, ((0, 0), (0, 0), (0, rp - r)))
    idx4 = idx.reshape(P, K, rp // CH, 2, SUB)
    z = _pool(table, idx4, rp, P, K)
    return _tc_readout(z, w2, b1v, b2v, r)


def kernel(x, edge_index, bonds, angles, propers, impropers, Ws1, Wn1, b1,
           Ws2, Wn2, b2, Wa1, ba1, Wa2, ba2, Wb1, bb1, Wb2, bb2, Wg1, bg1,
           Wg2, bg2, Wp1, bp1, Wp2, bp2, Wi1, bi1, Wi2, bi2):
    n = x.shape[0]
    e = edge_index.shape[1]
    n_pad = _ceil_to(n + 1, NS * 8)
    e_pad = _ceil_to(e, SEG_G * SUB * NW)

    src = jnp.pad(edge_index[0], (0, e_pad - e))
    dst = jnp.pad(edge_index[1], (0, e_pad - e), constant_values=n)
    n_groups = e_pad // (SEG_G * SUB * NW)
    src4 = src.reshape(n_groups, NW, SEG_G, SUB)
    dst4 = dst.reshape(n_groups, NW, SEG_G, SUB)

    agg1 = _segsum(x, src4, dst4, n_pad)[:, :n]
    h = _tc_layer(x, agg1, Ws1, Wn1, b1)
    agg2 = _segsum(h, src4, dst4, n_pad)[:, :n]
    atoms, tb, tg, tp, ti = _tc_heads(h, agg2, Ws2, Wn2, b2, Wa1, ba1, Wa2,
                                      ba2, Wb1, Wg1, Wp1, Wi1)

    bonds_out = _readout(tb.reshape(2 * n, 128), bonds, [(0, 1), (1, 0)],
                         n, Wb2, bb1, bb2)
    angles_out = _readout(tg.reshape(3 * n, 128), angles,
                          [(0, 1, 2), (2, 1, 0)], n, Wg2, bg1, bg2)
    propers_out = _readout(tp.reshape(4 * n, 128), propers,
                           [(0, 1, 2, 3), (3, 2, 1, 0)], n, Wp2, bp1, bp2)
    imp_perms = [(0, 1, 2, 3), (2, 1, 3, 0), (3, 1, 0, 2)]
    impropers_out = _readout(ti.reshape(4 * n, 128), impropers, imp_perms,
                             n, Wi2, bi1, bi2)
    return (atoms, bonds_out, angles_out, propers_out, impropers_out)
